# baseline (speedup 1.0000x reference)
import jax
import jax.numpy as jnp
from jax import lax
from jax.experimental import pallas as pl
from jax.experimental.pallas import tpu as pltpu
from jax.experimental.pallas import tpu_sc as plsc

_NCELLS = 14
_TPTS = 100000
_BATCH = 16384
_LANES = 16
_BH = _BATCH // 2
_QTR = _BH // 2
_UNROLL = 4


def _gather_body(released_hbm, if_hbm, out_hbm, row_v, buf_v, sem):
    c = lax.axis_index("c")
    s = lax.axis_index("s")
    wid = s * 2 + c

    @pl.when(wid < 2 * _NCELLS)
    def _():
        cell = wid // 2
        bh = wid % 2
        row_cp = pltpu.async_copy(released_hbm.at[cell], row_v, sem)
        idx_cp = pltpu.async_copy(if_hbm.at[pl.ds(bh * _BH, _BH)], buf_v, sem)
        row_cp.wait()
        idx_cp.wait()

        out_row = out_hbm.at[cell]
        half_cps = []
        for h in range(2):
            @plsc.parallel_loop(0, _QTR, step=_LANES, unroll=_UNROLL)
            def _gather(off):
                pos = h * _QTR + off
                idx = plsc.bitcast(buf_v[pl.ds(pos, _LANES)], jnp.int32)
                buf_v[pl.ds(pos, _LANES)] = plsc.load_gather(row_v, [idx])

            half_cps.append(
                pltpu.async_copy(
                    buf_v.at[pl.ds(h * _QTR, _QTR)],
                    out_row.at[pl.ds(bh * _BH + h * _QTR, _QTR)],
                    sem,
                )
            )
        for cp in half_cps:
            cp.wait()


def kernel(released, i):
    mesh = plsc.VectorSubcoreMesh(
        core_axis_name="c", subcore_axis_name="s", num_cores=2, num_subcores=16
    )
    f = pl.kernel(
        _gather_body,
        out_type=jax.ShapeDtypeStruct((_NCELLS, _BATCH), jnp.float32),
        mesh=mesh,
        compiler_params=pltpu.CompilerParams(
            needs_layout_passes=False,
            disable_bounds_checks=True,
            skip_device_barrier=True,
        ),
        scratch_types=[
            pltpu.VMEM((_TPTS,), jnp.float32),
            pltpu.VMEM((_BH,), jnp.float32),
            pltpu.SemaphoreType.DMA,
        ],
    )
    i_f = lax.bitcast_convert_type(i.astype(jnp.int32), jnp.float32)
    return f(released, i_f)
